# baseline (device time: 271100 ns/iter reference)
import jax
import jax.numpy as jnp
from jax import lax
from jax.experimental import pallas as pl
from jax.experimental.pallas import tpu as pltpu

N_DEV = 8
SQ = 2048
SKV_LOCAL = 2048
HQ = 8
DH = 128
DM = HQ * DH
QBLK = 512
SCALE = 0.08838834764831843


def kernel(x, Wq, K_ext, V_ext, Wo):
    x2 = x[0].astype(jnp.bfloat16)
    wq = Wq.astype(jnp.bfloat16)
    k2 = K_ext[0].reshape(SKV_LOCAL, DM).astype(jnp.bfloat16)
    v2 = V_ext[0].reshape(SKV_LOCAL, DM).astype(jnp.bfloat16)
    wo = Wo.astype(jnp.bfloat16)

    def body(x_ref, wq_ref, k_ref, v_ref, wo_ref, out_ref,
             ctx_ref, send_sem, recv_sem):
        my = lax.axis_index("i")

        @pl.when(my == 0)
        def _():
            q = jnp.dot(
                x_ref[...], wq_ref[...], preferred_element_type=jnp.float32
            ).astype(jnp.bfloat16)
            for h in range(HQ):
                kh = k_ref[:, h * DH:(h + 1) * DH]
                vh = v_ref[:, h * DH:(h + 1) * DH]
                for b in range(SQ // QBLK):
                    qh = q[b * QBLK:(b + 1) * QBLK, h * DH:(h + 1) * DH]
                    s = lax.dot_general(
                        qh, kh, (((1,), (1,)), ((), ())),
                        preferred_element_type=jnp.float32,
                    ) * SCALE
                    rows = b * QBLK + lax.broadcasted_iota(
                        jnp.int32, (QBLK, SKV_LOCAL), 0)
                    cols = lax.broadcasted_iota(
                        jnp.int32, (QBLK, SKV_LOCAL), 1)
                    mask = (cols // 64) <= (rows // 64)
                    s = jnp.where(mask, s, -1e9)
                    m = jnp.max(s, axis=-1, keepdims=True)
                    p = jnp.exp(s - m)
                    l = jnp.sum(p, axis=-1, keepdims=True)
                    p = (p / l).astype(jnp.bfloat16)
                    ctx_ref[b * QBLK:(b + 1) * QBLK, h * DH:(h + 1) * DH] = (
                        jnp.dot(p, vh, preferred_element_type=jnp.float32)
                        .astype(jnp.bfloat16)
                    )

        @pl.when(my != 0)
        def _():
            recv = pltpu.make_async_remote_copy(
                src_ref=ctx_ref, dst_ref=ctx_ref,
                send_sem=send_sem, recv_sem=recv_sem,
                device_id=(my - 1,), device_id_type=pl.DeviceIdType.MESH,
            )
            recv.wait_recv()

        @pl.when(my != N_DEV - 1)
        def _():
            send = pltpu.make_async_remote_copy(
                src_ref=ctx_ref, dst_ref=ctx_ref,
                send_sem=send_sem, recv_sem=recv_sem,
                device_id=(my + 1,), device_id_type=pl.DeviceIdType.MESH,
            )
            send.start()
            send.wait_send()

        out_ref[...] = jnp.dot(
            ctx_ref[...], wo_ref[...], preferred_element_type=jnp.float32
        )

    out = pl.pallas_call(
        body,
        out_shape=jax.ShapeDtypeStruct((SQ, DM), jnp.float32),
        in_specs=[pl.BlockSpec(memory_space=pltpu.VMEM)] * 5,
        out_specs=pl.BlockSpec(memory_space=pltpu.VMEM),
        scratch_shapes=[
            pltpu.VMEM((SQ, DM), jnp.bfloat16),
            pltpu.SemaphoreType.DMA,
            pltpu.SemaphoreType.DMA,
        ],
    )(x2, wq, k2, v2, wo)
    return out.reshape(1, SQ, DM)


# device time: 112892 ns/iter; 2.4014x vs baseline; 2.4014x over previous
import jax
import jax.numpy as jnp
from jax import lax
from jax.experimental import pallas as pl
from jax.experimental.pallas import tpu as pltpu

N_DEV = 8
SQ = 2048
SKV_LOCAL = 2048
HQ = 8
DH = 128
DM = HQ * DH
QBLK = 512
NCHUNK = SQ // QBLK
SCALE = 0.08838834764831843


def kernel(x, Wq, K_ext, V_ext, Wo):
    x2 = x[0].astype(jnp.bfloat16)
    wq = Wq.astype(jnp.bfloat16)
    k2 = K_ext[0].reshape(SKV_LOCAL, DM).astype(jnp.bfloat16)
    v2 = V_ext[0].reshape(SKV_LOCAL, DM).astype(jnp.bfloat16)
    wo = Wo.astype(jnp.bfloat16)

    def body(x_ref, wq_ref, k_ref, v_ref, wo_ref, out_ref,
             ctx_ref, send_sems, recv_sems):
        my = lax.axis_index("i")
        right = (my + 1) % N_DEV
        left = (my - 1) % N_DEV

        barrier_sem = pltpu.get_barrier_semaphore()
        for nbr in (left, right):
            pl.semaphore_signal(
                barrier_sem, inc=1,
                device_id=(nbr,), device_id_type=pl.DeviceIdType.MESH,
            )
        pl.semaphore_wait(barrier_sem, 2)

        def chunk_rdma(b, target, sem_row):
            return pltpu.make_async_remote_copy(
                src_ref=ctx_ref.at[pl.ds(b * QBLK, QBLK)],
                dst_ref=ctx_ref.at[pl.ds(b * QBLK, QBLK)],
                send_sem=send_sems.at[sem_row, b],
                recv_sem=recv_sems.at[b],
                device_id=(target,), device_id_type=pl.DeviceIdType.MESH,
            )

        def project(b):
            out_ref[b * QBLK:(b + 1) * QBLK, :] = jnp.dot(
                ctx_ref[b * QBLK:(b + 1) * QBLK, :], wo_ref[...],
                preferred_element_type=jnp.float32,
            )

        @pl.when(my == 0)
        def _():
            for b in range(NCHUNK):
                kmax = (b + 1) * QBLK
                qb = jnp.dot(
                    x_ref[b * QBLK:(b + 1) * QBLK, :], wq_ref[...],
                    preferred_element_type=jnp.float32,
                ).astype(jnp.bfloat16)
                for h in range(HQ):
                    kh = k_ref[:kmax, h * DH:(h + 1) * DH]
                    vh = v_ref[:kmax, h * DH:(h + 1) * DH]
                    qh = qb[:, h * DH:(h + 1) * DH]
                    s = lax.dot_general(
                        qh, kh, (((1,), (1,)), ((), ())),
                        preferred_element_type=jnp.float32,
                    ) * SCALE
                    rows = b * QBLK + lax.broadcasted_iota(
                        jnp.int32, (QBLK, kmax), 0)
                    cols = lax.broadcasted_iota(jnp.int32, (QBLK, kmax), 1)
                    s = jnp.where((cols // 64) <= (rows // 64), s, -1e9)
                    m = jnp.max(s, axis=-1, keepdims=True)
                    p = jnp.exp(s - m)
                    l = jnp.sum(p, axis=-1, keepdims=True)
                    p = (p / l).astype(jnp.bfloat16)
                    ctx_ref[b * QBLK:(b + 1) * QBLK, h * DH:(h + 1) * DH] = (
                        jnp.dot(p, vh, preferred_element_type=jnp.float32)
                        .astype(jnp.bfloat16)
                    )
                chunk_rdma(b, 1, 0).start()
                chunk_rdma(b, N_DEV - 1, 1).start()
                project(b)
            for b in range(NCHUNK):
                chunk_rdma(b, 1, 0).wait_send()
                chunk_rdma(b, N_DEV - 1, 1).wait_send()

        @pl.when((my >= 1) & (my <= 3))
        def _():
            for b in range(NCHUNK):
                chunk_rdma(b, left, 0).wait_recv()

                @pl.when(my < 3)
                def _():
                    chunk_rdma(b, right, 0).start()

                project(b)

            @pl.when(my < 3)
            def _():
                for b in range(NCHUNK):
                    chunk_rdma(b, right, 0).wait_send()

        @pl.when(my >= 4)
        def _():
            for b in range(NCHUNK):
                chunk_rdma(b, right, 0).wait_recv()

                @pl.when(my > 4)
                def _():
                    chunk_rdma(b, left, 0).start()

                project(b)

            @pl.when(my > 4)
            def _():
                for b in range(NCHUNK):
                    chunk_rdma(b, left, 0).wait_send()

    out = pl.pallas_call(
        body,
        out_shape=jax.ShapeDtypeStruct((SQ, DM), jnp.float32),
        in_specs=[pl.BlockSpec(memory_space=pltpu.VMEM)] * 5,
        out_specs=pl.BlockSpec(memory_space=pltpu.VMEM),
        scratch_shapes=[
            pltpu.VMEM((SQ, DM), jnp.bfloat16),
            pltpu.SemaphoreType.DMA((2, NCHUNK)),
            pltpu.SemaphoreType.DMA((NCHUNK,)),
        ],
        compiler_params=pltpu.CompilerParams(collective_id=0),
    )(x2, wq, k2, v2, wo)
    return out.reshape(1, SQ, DM)


# device time: 64501 ns/iter; 4.2030x vs baseline; 1.7502x over previous
import jax
import jax.numpy as jnp
from jax import lax
from jax.experimental import pallas as pl
from jax.experimental.pallas import tpu as pltpu

SQ = 2048
SKV_LOCAL = 2048
HQ = 8
DH = 128
DM = HQ * DH
QBLK = 512
SCALE = 0.08838834764831843


def kernel(x, Wq, K_ext, V_ext, Wo):
    x2 = x[0].astype(jnp.bfloat16)
    wq = Wq.astype(jnp.bfloat16)
    k2 = K_ext[0].reshape(SKV_LOCAL, DM).astype(jnp.bfloat16)
    v2 = V_ext[0].reshape(SKV_LOCAL, DM).astype(jnp.bfloat16)
    wo = Wo.astype(jnp.bfloat16)

    def body(x_ref, wq_ref, k_ref, v_ref, wo_ref, out_ref, ctx_ref):
        for b in range(SQ // QBLK):
            kmax = (b + 1) * QBLK
            qb = jnp.dot(
                x_ref[b * QBLK:(b + 1) * QBLK, :], wq_ref[...],
                preferred_element_type=jnp.float32,
            ).astype(jnp.bfloat16)
            for h in range(HQ):
                kh = k_ref[:kmax, h * DH:(h + 1) * DH]
                vh = v_ref[:kmax, h * DH:(h + 1) * DH]
                qh = qb[:, h * DH:(h + 1) * DH]
                s = lax.dot_general(
                    qh, kh, (((1,), (1,)), ((), ())),
                    preferred_element_type=jnp.float32,
                ) * SCALE
                rows = b * QBLK + lax.broadcasted_iota(jnp.int32, (QBLK, kmax), 0)
                cols = lax.broadcasted_iota(jnp.int32, (QBLK, kmax), 1)
                s = jnp.where((cols // 64) <= (rows // 64), s, -1e9)
                m = jnp.max(s, axis=-1, keepdims=True)
                p = jnp.exp(s - m)
                l = jnp.sum(p, axis=-1, keepdims=True)
                p = (p / l).astype(jnp.bfloat16)
                ctx_ref[b * QBLK:(b + 1) * QBLK, h * DH:(h + 1) * DH] = (
                    jnp.dot(p, vh, preferred_element_type=jnp.float32)
                    .astype(jnp.bfloat16)
                )
            out_ref[b * QBLK:(b + 1) * QBLK, :] = jnp.dot(
                ctx_ref[b * QBLK:(b + 1) * QBLK, :], wo_ref[...],
                preferred_element_type=jnp.float32,
            )

    out = pl.pallas_call(
        body,
        out_shape=jax.ShapeDtypeStruct((SQ, DM), jnp.float32),
        in_specs=[pl.BlockSpec(memory_space=pltpu.VMEM)] * 5,
        out_specs=pl.BlockSpec(memory_space=pltpu.VMEM),
        scratch_shapes=[pltpu.VMEM((SQ, DM), jnp.bfloat16)],
    )(x2, wq, k2, v2, wo)
    return out.reshape(1, SQ, DM)
